# baseline (speedup 1.0000x reference)
import functools

import jax
import jax.numpy as jnp
from jax import lax
from jax.experimental import pallas as pl
from jax.experimental.pallas import tpu as pltpu
from jax.experimental.pallas import tpu_sc as plsc

N = 10000
E = 320000
D_IN = 128
D_HID = 128
D_OUT = 2
EPS = 1e-5

NC = 2
NS = 16
CH = 128
NCHUNK = 81
EPT = CH * NCHUNK
EP = EPT * NC * NS
DUMMY = N
NPAD = 10016
ZROWS = NPAD // NS
OROWS = 624
OTAIL = N - OROWS * NS

_f32 = jnp.float32
_mesh = plsc.VectorSubcoreMesh(core_axis_name="c", subcore_axis_name="s")


def _make_prop(width, nbuf, ring):
  ng = NCHUNK // nbuf
  assert ng * nbuf == NCHUNK
  idx_shape = (2, nbuf, CH) if ring else (NCHUNK, CH)

  @functools.partial(
      pl.kernel,
      out_type=jax.ShapeDtypeStruct((NC, N, width), _f32),
      mesh=_mesh,
      compiler_params=pltpu.CompilerParams(use_tc_tiling_on_sc=False),
      scratch_types=[
          pltpu.VMEM_SHARED((NPAD, width), _f32),
          pltpu.VMEM(idx_shape, jnp.int32),
          pltpu.VMEM(idx_shape, jnp.int32),
          [pltpu.VMEM((CH, width), _f32)] * nbuf,
          [pltpu.SemaphoreType.DMA] * nbuf,
          [pltpu.SemaphoreType.DMA] * nbuf,
          [pltpu.SemaphoreType.DMA] * 2,
      ],
  )
  def prop(src_hbm, dst_hbm, rows_hbm, zeros_hbm, out_hbm,
           acc, srcb, dstb, rows, gsem, ssem, isem):
    c = lax.axis_index("c")
    s = lax.axis_index("s")
    wid = c * NS + s
    grp0 = wid * NCHUNK

    def src_row(j, r):
      return srcb.at[lax.rem(j, 2), r] if ring else srcb.at[j * nbuf + r]

    def dst_row(j, r):
      return dstb.at[lax.rem(j, 2), r] if ring else dstb.at[j * nbuf + r]

    if ring:
      pltpu.sync_copy(src_hbm.at[pl.ds(grp0, nbuf)], srcb.at[0])
      pltpu.sync_copy(dst_hbm.at[pl.ds(grp0, nbuf)], dstb.at[0])
    else:
      pltpu.sync_copy(src_hbm.at[pl.ds(grp0, NCHUNK)], srcb)
      pltpu.sync_copy(dst_hbm.at[pl.ds(grp0, NCHUNK)], dstb)
    pltpu.sync_copy(zeros_hbm, acc.at[pl.ds(s * ZROWS, ZROWS)])
    plsc.subcore_barrier()

    for r in range(nbuf):
      pltpu.async_copy(rows_hbm.at[src_row(0, r)], rows[r], gsem[r])

    def group(j, carry):
      q = 1 - lax.rem(j, 2)
      if ring:
        @pl.when(j < ng - 1)
        def _():
          nxt = grp0 + (j + 1) * nbuf
          pltpu.async_copy(src_hbm.at[pl.ds(nxt, nbuf)], srcb.at[q], isem[0])
          pltpu.async_copy(dst_hbm.at[pl.ds(nxt, nbuf)], dstb.at[q], isem[1])

      for r in range(nbuf):
        pltpu.make_async_copy(rows_hbm.at[src_row(j, r)],
                              rows[r], gsem[r]).wait()
        pltpu.async_copy(rows[r], acc.at[dst_row(j, r)], ssem[r], add=True)

      if ring:
        @pl.when(j < ng - 1)
        def _():
          nxt = grp0 + (j + 1) * nbuf
          pltpu.make_async_copy(src_hbm.at[pl.ds(nxt, nbuf)], srcb.at[q],
                                isem[0]).wait()
          pltpu.make_async_copy(dst_hbm.at[pl.ds(nxt, nbuf)], dstb.at[q],
                                isem[1]).wait()

      for r in range(nbuf):
        pltpu.make_async_copy(rows[r], acc.at[dst_row(j, r)],
                              ssem[r]).wait()

        @pl.when(j < ng - 1)
        def _():
          pltpu.async_copy(rows_hbm.at[src_row(j + 1, r)], rows[r], gsem[r])
      return carry

    lax.fori_loop(0, ng, group, 0)
    plsc.subcore_barrier()
    pltpu.sync_copy(acc.at[pl.ds(s * OROWS, OROWS)],
                    out_hbm.at[c, pl.ds(s * OROWS, OROWS)])

    @pl.when(s == NS - 1)
    def _():
      pltpu.sync_copy(acc.at[pl.ds(NS * OROWS, OTAIL)],
                      out_hbm.at[c, pl.ds(NS * OROWS, OTAIL)])

  return prop


@functools.partial(
    pl.kernel,
    out_type=jax.ShapeDtypeStruct((NC, N, 8), _f32),
    mesh=_mesh,
    compiler_params=pltpu.CompilerParams(use_tc_tiling_on_sc=False),
    scratch_types=[
        pltpu.VMEM_SHARED((NPAD, 8), _f32),
        pltpu.VMEM((NCHUNK, CH), jnp.int32),
        pltpu.VMEM((CH, 8), _f32),
        [pltpu.SemaphoreType.DMA] * 9,
    ],
)
def _deg_kernel(dst_hbm, ones_hbm, zeros_hbm, out_hbm, acc, dstb, ones_v,
                ssem):
  nbuf = 9
  ng = NCHUNK // nbuf
  c = lax.axis_index("c")
  s = lax.axis_index("s")
  wid = c * NS + s
  pltpu.sync_copy(dst_hbm.at[pl.ds(wid * NCHUNK, NCHUNK)], dstb)
  pltpu.sync_copy(zeros_hbm, acc.at[pl.ds(s * ZROWS, ZROWS)])
  pltpu.sync_copy(ones_hbm, ones_v)
  plsc.subcore_barrier()

  def group(j, carry):
    k0 = j * nbuf
    for r in range(nbuf):

      @pl.when(j > 0)
      def _():
        pltpu.make_async_copy(ones_v, acc.at[dstb.at[k0 - nbuf + r]],
                              ssem[r]).wait()

      pltpu.async_copy(ones_v, acc.at[dstb.at[k0 + r]], ssem[r], add=True)
    return carry

  lax.fori_loop(0, ng, group, 0)
  for r in range(nbuf):
    pltpu.make_async_copy(ones_v, acc.at[dstb.at[(ng - 1) * nbuf + r]],
                          ssem[r]).wait()
  plsc.subcore_barrier()
  pltpu.sync_copy(acc.at[pl.ds(s * OROWS, OROWS)],
                  out_hbm.at[c, pl.ds(s * OROWS, OROWS)])

  @pl.when(s == NS - 1)
  def _():
    pltpu.sync_copy(acc.at[pl.ds(NS * OROWS, OTAIL)],
                    out_hbm.at[c, pl.ds(NS * OROWS, OTAIL)])


_prop128 = _make_prop(D_HID, 3, True)
_prop8 = _make_prop(8, 9, False)

BLK = 1000
GRID = N // BLK


def _dinv_of(deg_ref):
  deg = deg_ref[0, :, 0] + deg_ref[1, :, 0] + 1.0
  return lax.rsqrt(deg)


def _prep1_body(deg_ref, u_ref, w1_ref, g1_ref):
  dinv = _dinv_of(deg_ref)
  h = jnp.dot(u_ref[...], w1_ref[...], preferred_element_type=_f32)
  g1_ref[...] = h * dinv[:, None]


def _s1_block(deg_ref, acc_ref, g1_ref, b1_ref):
  dinv = _dinv_of(deg_ref)
  s1 = dinv[:, None] * (acc_ref[0] + acc_ref[1] + g1_ref[...]) + b1_ref[...]
  return dinv, s1


def _stats1_body(deg_ref, acc_ref, g1_ref, b1_ref, st_ref):
  _, s1 = _s1_block(deg_ref, acc_ref, g1_ref, b1_ref)
  pid = pl.program_id(0)

  @pl.when(pid == 0)
  def _():
    st_ref[...] = jnp.zeros_like(st_ref)

  upd = jnp.concatenate(
      [jnp.sum(s1, axis=0)[None], jnp.sum(s1 * s1, axis=0)[None],
       jnp.zeros((6, s1.shape[1]), _f32)], axis=0)
  st_ref[...] += upd


def _apply1_body(deg_ref, acc_ref, g1_ref, b1_ref, st_ref, gam_ref, bet_ref,
                 w2_ref, g2_ref):
  dinv, s1 = _s1_block(deg_ref, acc_ref, g1_ref, b1_ref)
  mean = st_ref[0:1, :] / N
  var = st_ref[1:2, :] / N - mean * mean
  xh = (s1 - mean) * lax.rsqrt(var + EPS)
  r = jnp.maximum(gam_ref[...] * xh + bet_ref[...], 0.0)
  h2 = jnp.dot(r, w2_ref[...], preferred_element_type=_f32)
  g2_ref[...] = h2 * dinv[:, None]


def _s2_block(deg_ref, acc_ref, g2_ref, b2_ref):
  dinv = _dinv_of(deg_ref)
  return dinv[:, None] * (acc_ref[0] + acc_ref[1] + g2_ref[...]) + b2_ref[...]


def _stats2_body(deg_ref, acc_ref, g2_ref, b2_ref, st_ref):
  s2 = _s2_block(deg_ref, acc_ref, g2_ref, b2_ref)
  pid = pl.program_id(0)

  @pl.when(pid == 0)
  def _():
    st_ref[...] = jnp.zeros_like(st_ref)

  upd = jnp.concatenate(
      [jnp.sum(s2, axis=0)[None], jnp.sum(s2 * s2, axis=0)[None],
       jnp.zeros((6, s2.shape[1]), _f32)], axis=0)
  st_ref[...] += upd


def _apply2_body(deg_ref, acc_ref, g2_ref, b2_ref, st_ref, gam_ref, bet_ref,
                 sig_ref, sm_ref):
  s2 = _s2_block(deg_ref, acc_ref, g2_ref, b2_ref)
  mean = st_ref[0:1, :] / N
  var = st_ref[1:2, :] / N - mean * mean
  xh = (s2 - mean) * lax.rsqrt(var + EPS)
  y = gam_ref[...] * xh + bet_ref[...]
  sig_ref[...] = 1.0 / (1.0 + jnp.exp(-y))
  a = y[:, 0:1]
  b = y[:, 1:2]
  m = jnp.maximum(a, b)
  ea = jnp.exp(a - m)
  eb = jnp.exp(b - m)
  tot = ea + eb
  sm_ref[...] = jnp.concatenate(
      [ea / tot, eb / tot, jnp.zeros((y.shape[0], 6), _f32)], axis=1)


def _row_spec(width):
  return pl.BlockSpec((BLK, width), lambda i: (i, 0))


_deg_spec = pl.BlockSpec((NC, BLK, 8), lambda i: (0, i, 0))
_full = lambda shape: pl.BlockSpec(shape, lambda i: tuple(0 for _ in shape))

_prep1 = pl.pallas_call(
    _prep1_body,
    grid=(GRID,),
    in_specs=[_deg_spec, _row_spec(D_IN), _full((D_IN, D_HID))],
    out_specs=_row_spec(D_HID),
    out_shape=jax.ShapeDtypeStruct((N, D_HID), _f32),
)

_stats1 = pl.pallas_call(
    _stats1_body,
    grid=(GRID,),
    in_specs=[_deg_spec, pl.BlockSpec((NC, BLK, D_HID), lambda i: (0, i, 0)),
              _row_spec(D_HID), _full((1, D_HID))],
    out_specs=_full((8, D_HID)),
    out_shape=jax.ShapeDtypeStruct((8, D_HID), _f32),
)

_apply1 = pl.pallas_call(
    _apply1_body,
    grid=(GRID,),
    in_specs=[_deg_spec, pl.BlockSpec((NC, BLK, D_HID), lambda i: (0, i, 0)),
              _row_spec(D_HID), _full((1, D_HID)), _full((8, D_HID)),
              _full((1, D_HID)), _full((1, D_HID)), _full((D_HID, 8))],
    out_specs=_row_spec(8),
    out_shape=jax.ShapeDtypeStruct((N, 8), _f32),
)

_stats2 = pl.pallas_call(
    _stats2_body,
    grid=(GRID,),
    in_specs=[_deg_spec, pl.BlockSpec((NC, BLK, 8), lambda i: (0, i, 0)),
              _row_spec(8), _full((1, 8))],
    out_specs=_full((8, 8)),
    out_shape=jax.ShapeDtypeStruct((8, 8), _f32),
)

_apply2 = pl.pallas_call(
    _apply2_body,
    grid=(GRID,),
    in_specs=[_deg_spec, pl.BlockSpec((NC, BLK, 8), lambda i: (0, i, 0)),
              _row_spec(8), _full((1, 8)), _full((8, 8)),
              _full((1, 8)), _full((1, 8))],
    out_specs=[_row_spec(8), _row_spec(8)],
    out_shape=[jax.ShapeDtypeStruct((N, 8), _f32),
               jax.ShapeDtypeStruct((N, 8), _f32)],
)


@jax.jit
def kernel(edge_index, u_S, W1, b1, gamma1, beta1, W2, b2, gamma2, beta2):
  pad = EP - E
  src = jnp.concatenate(
      [edge_index[0], jnp.zeros((pad,), jnp.int32)]).reshape(EP // CH, CH)
  dst = jnp.concatenate(
      [edge_index[1], jnp.full((pad,), DUMMY, jnp.int32)]).reshape(EP // CH, CH)

  ones8 = jnp.ones((CH, 8), _f32)
  zeros8 = jnp.zeros((ZROWS, 8), _f32)
  zeros128 = jnp.zeros((ZROWS, D_HID), _f32)

  deg2 = _deg_kernel(dst, ones8, zeros8)

  w1 = W1.astype(_f32)
  g1 = _prep1(deg2, u_S, w1)
  acc1 = _prop128(src, dst, g1, zeros128)

  b1r = b1.reshape(1, D_HID)
  st1 = _stats1(deg2, acc1, g1, b1r)
  w2p = jnp.concatenate([W2, jnp.zeros((D_HID, 8 - D_OUT), _f32)], axis=1)
  g2 = _apply1(deg2, acc1, g1, b1r, st1, gamma1.reshape(1, -1),
               beta1.reshape(1, -1), w2p)

  acc2 = _prop8(src, dst, g2, zeros8)

  pad2 = lambda v: jnp.concatenate([v, jnp.zeros((8 - D_OUT,), _f32)]).reshape(1, 8)
  b2r = pad2(b2)
  st2 = _stats2(deg2, acc2, g2, b2r)
  sig, sm = _apply2(deg2, acc2, g2, b2r, st2, pad2(gamma2), pad2(beta2))
  return sig[:, :D_OUT], sm[:, :D_OUT]


# speedup vs baseline: 3.5259x; 3.5259x over previous
import functools

import jax
import jax.numpy as jnp
from jax import lax
from jax.experimental import pallas as pl
from jax.experimental.pallas import tpu as pltpu
from jax.experimental.pallas import tpu_sc as plsc

N = 10000
E = 320000
D_IN = 128
D_HID = 128
D_OUT = 2
EPS = 1e-5

NC = 2
NS = 16
CH = 128
NCHUNK = 81
EPT = CH * NCHUNK
EP = EPT * NC * NS
RC = E // CH
NPAD = N
ZROWS = 632
ZTAIL = NPAD - 15 * ZROWS
OROWS = 624
OTAIL = N - OROWS * NS

_f32 = jnp.float32
_mesh = plsc.VectorSubcoreMesh(core_axis_name="c", subcore_axis_name="s")


def _make_prop(width, nbuf, ring):
  ng = NCHUNK // nbuf
  assert ng * nbuf == NCHUNK
  idx_shape = (2, nbuf, CH) if ring else (NCHUNK, CH)

  @functools.partial(
      pl.kernel,
      out_type=jax.ShapeDtypeStruct((NC, N, width), _f32),
      mesh=_mesh,
      compiler_params=pltpu.CompilerParams(use_tc_tiling_on_sc=False),
      scratch_types=[
          pltpu.VMEM_SHARED((NPAD, width), _f32),
          pltpu.VMEM(idx_shape, jnp.int32),
          pltpu.VMEM(idx_shape, jnp.int32),
          [pltpu.VMEM((CH, width), _f32)] * nbuf,
          [pltpu.SemaphoreType.DMA] * nbuf,
          [pltpu.SemaphoreType.DMA] * nbuf,
          [pltpu.SemaphoreType.DMA] * 2,
      ],
  )
  def prop(src_hbm, dst_hbm, rows_hbm, zeros_hbm, out_hbm,
           acc, srcb, dstb, rows, gsem, ssem, isem):
    c = lax.axis_index("c")
    s = lax.axis_index("s")
    wid = c * NS + s
    grp0 = wid * NCHUNK

    def src_row(j, r):
      return srcb.at[lax.rem(j, 2), r] if ring else srcb.at[j * nbuf + r]

    def dst_row(j, r):
      return dstb.at[lax.rem(j, 2), r] if ring else dstb.at[j * nbuf + r]

    if ring:
      pltpu.sync_copy(src_hbm.at[pl.ds(grp0, nbuf)], srcb.at[0])
      pltpu.sync_copy(dst_hbm.at[pl.ds(grp0, nbuf)], dstb.at[0])
    else:
      pltpu.sync_copy(src_hbm.at[pl.ds(grp0, NCHUNK)], srcb)
      pltpu.sync_copy(dst_hbm.at[pl.ds(grp0, NCHUNK)], dstb)

    @pl.when(s < NS - 1)
    def _():
      pltpu.sync_copy(zeros_hbm, acc.at[pl.ds(s * ZROWS, ZROWS)])

    @pl.when(s == NS - 1)
    def _():
      pltpu.sync_copy(zeros_hbm.at[pl.ds(0, ZTAIL)],
                      acc.at[pl.ds((NS - 1) * ZROWS, ZTAIL)])

    plsc.subcore_barrier()

    for r in range(nbuf):

      @pl.when(grp0 + r < RC)
      def _():
        pltpu.async_copy(rows_hbm.at[src_row(0, r)], rows[r], gsem[r])

    def group(j, carry):
      q = 1 - lax.rem(j, 2)
      if ring:
        @pl.when(j < ng - 1)
        def _():
          nxt = grp0 + (j + 1) * nbuf
          pltpu.async_copy(src_hbm.at[pl.ds(nxt, nbuf)], srcb.at[q], isem[0])
          pltpu.async_copy(dst_hbm.at[pl.ds(nxt, nbuf)], dstb.at[q], isem[1])

      for r in range(nbuf):
        kg = grp0 + j * nbuf + r

        @pl.when(kg < RC)
        def _():
          pltpu.make_async_copy(rows_hbm.at[src_row(j, r)],
                                rows[r], gsem[r]).wait()
          pltpu.async_copy(rows[r], acc.at[dst_row(j, r)], ssem[r], add=True)

      if ring:
        @pl.when(j < ng - 1)
        def _():
          nxt = grp0 + (j + 1) * nbuf
          pltpu.make_async_copy(src_hbm.at[pl.ds(nxt, nbuf)], srcb.at[q],
                                isem[0]).wait()
          pltpu.make_async_copy(dst_hbm.at[pl.ds(nxt, nbuf)], dstb.at[q],
                                isem[1]).wait()

      for r in range(nbuf):
        kg = grp0 + j * nbuf + r

        @pl.when(kg < RC)
        def _():
          pltpu.make_async_copy(rows[r], acc.at[dst_row(j, r)],
                                ssem[r]).wait()

        @pl.when(jnp.logical_and(j < ng - 1, kg + nbuf < RC))
        def _():
          pltpu.async_copy(rows_hbm.at[src_row(j + 1, r)], rows[r], gsem[r])
      return carry

    lax.fori_loop(0, ng, group, 0)
    plsc.subcore_barrier()
    pltpu.sync_copy(acc.at[pl.ds(s * OROWS, OROWS)],
                    out_hbm.at[c, pl.ds(s * OROWS, OROWS)])

    @pl.when(s == NS - 1)
    def _():
      pltpu.sync_copy(acc.at[pl.ds(NS * OROWS, OTAIL)],
                      out_hbm.at[c, pl.ds(NS * OROWS, OTAIL)])

  return prop


@functools.partial(
    pl.kernel,
    out_type=jax.ShapeDtypeStruct((NC, N, 8), _f32),
    mesh=_mesh,
    compiler_params=pltpu.CompilerParams(use_tc_tiling_on_sc=False),
    scratch_types=[
        pltpu.VMEM_SHARED((NPAD, 8), _f32),
        pltpu.VMEM((NCHUNK, CH), jnp.int32),
        pltpu.VMEM((CH, 8), _f32),
        [pltpu.SemaphoreType.DMA] * 9,
    ],
)
def _deg_kernel(dst_hbm, ones_hbm, zeros_hbm, out_hbm, acc, dstb, ones_v,
                ssem):
  nbuf = 9
  ng = NCHUNK // nbuf
  c = lax.axis_index("c")
  s = lax.axis_index("s")
  wid = c * NS + s
  pltpu.sync_copy(dst_hbm.at[pl.ds(wid * NCHUNK, NCHUNK)], dstb)

  @pl.when(s < NS - 1)
  def _():
    pltpu.sync_copy(zeros_hbm, acc.at[pl.ds(s * ZROWS, ZROWS)])

  @pl.when(s == NS - 1)
  def _():
    pltpu.sync_copy(zeros_hbm.at[pl.ds(0, ZTAIL)],
                    acc.at[pl.ds((NS - 1) * ZROWS, ZTAIL)])

  pltpu.sync_copy(ones_hbm, ones_v)
  plsc.subcore_barrier()

  grp0 = wid * NCHUNK

  def group(j, carry):
    k0 = j * nbuf
    for r in range(nbuf):

      @pl.when(jnp.logical_and(j > 0, grp0 + k0 - nbuf + r < RC))
      def _():
        pltpu.make_async_copy(ones_v, acc.at[dstb.at[k0 - nbuf + r]],
                              ssem[r]).wait()

      @pl.when(grp0 + k0 + r < RC)
      def _():
        pltpu.async_copy(ones_v, acc.at[dstb.at[k0 + r]], ssem[r], add=True)

    return carry

  lax.fori_loop(0, ng, group, 0)
  for r in range(nbuf):

    @pl.when(grp0 + (ng - 1) * nbuf + r < RC)
    def _():
      pltpu.make_async_copy(ones_v, acc.at[dstb.at[(ng - 1) * nbuf + r]],
                            ssem[r]).wait()

  plsc.subcore_barrier()
  pltpu.sync_copy(acc.at[pl.ds(s * OROWS, OROWS)],
                  out_hbm.at[c, pl.ds(s * OROWS, OROWS)])

  @pl.when(s == NS - 1)
  def _():
    pltpu.sync_copy(acc.at[pl.ds(NS * OROWS, OTAIL)],
                    out_hbm.at[c, pl.ds(NS * OROWS, OTAIL)])


_prop128 = _make_prop(D_HID, 3, True)
_prop8 = _make_prop(8, 9, False)

BLK = 1000
GRID = N // BLK


def _dinv_of(deg_ref):
  deg = deg_ref[0, :, 0] + deg_ref[1, :, 0] + 1.0
  return lax.rsqrt(deg)


def _prep1_body(deg_ref, u_ref, w1_ref, g1_ref):
  dinv = _dinv_of(deg_ref)
  h = jnp.dot(u_ref[...], w1_ref[...], preferred_element_type=_f32)
  g1_ref[...] = h * dinv[:, None]


def _s1_block(deg_ref, acc_ref, g1_ref, b1_ref):
  dinv = _dinv_of(deg_ref)
  s1 = dinv[:, None] * (acc_ref[0] + acc_ref[1] + g1_ref[...]) + b1_ref[...]
  return dinv, s1


def _stats1_body(deg_ref, acc_ref, g1_ref, b1_ref, st_ref):
  _, s1 = _s1_block(deg_ref, acc_ref, g1_ref, b1_ref)
  pid = pl.program_id(0)

  @pl.when(pid == 0)
  def _():
    st_ref[...] = jnp.zeros_like(st_ref)

  upd = jnp.concatenate(
      [jnp.sum(s1, axis=0)[None], jnp.sum(s1 * s1, axis=0)[None],
       jnp.zeros((6, s1.shape[1]), _f32)], axis=0)
  st_ref[...] += upd


def _apply1_body(deg_ref, acc_ref, g1_ref, b1_ref, st_ref, gam_ref, bet_ref,
                 w2_ref, g2_ref):
  dinv, s1 = _s1_block(deg_ref, acc_ref, g1_ref, b1_ref)
  mean = st_ref[0:1, :] / N
  var = st_ref[1:2, :] / N - mean * mean
  xh = (s1 - mean) * lax.rsqrt(var + EPS)
  r = jnp.maximum(gam_ref[...] * xh + bet_ref[...], 0.0)
  h2 = jnp.dot(r, w2_ref[...], preferred_element_type=_f32)
  g2_ref[...] = h2 * dinv[:, None]


def _s2_block(deg_ref, acc_ref, g2_ref, b2_ref):
  dinv = _dinv_of(deg_ref)
  return dinv[:, None] * (acc_ref[0] + acc_ref[1] + g2_ref[...]) + b2_ref[...]


def _stats2_body(deg_ref, acc_ref, g2_ref, b2_ref, st_ref):
  s2 = _s2_block(deg_ref, acc_ref, g2_ref, b2_ref)
  pid = pl.program_id(0)

  @pl.when(pid == 0)
  def _():
    st_ref[...] = jnp.zeros_like(st_ref)

  upd = jnp.concatenate(
      [jnp.sum(s2, axis=0)[None], jnp.sum(s2 * s2, axis=0)[None],
       jnp.zeros((6, s2.shape[1]), _f32)], axis=0)
  st_ref[...] += upd


def _apply2_body(deg_ref, acc_ref, g2_ref, b2_ref, st_ref, gam_ref, bet_ref,
                 sig_ref, sm_ref):
  s2 = _s2_block(deg_ref, acc_ref, g2_ref, b2_ref)
  mean = st_ref[0:1, :] / N
  var = st_ref[1:2, :] / N - mean * mean
  xh = (s2 - mean) * lax.rsqrt(var + EPS)
  y = gam_ref[...] * xh + bet_ref[...]
  sig_ref[...] = 1.0 / (1.0 + jnp.exp(-y))
  a = y[:, 0:1]
  b = y[:, 1:2]
  m = jnp.maximum(a, b)
  ea = jnp.exp(a - m)
  eb = jnp.exp(b - m)
  tot = ea + eb
  sm_ref[...] = jnp.concatenate(
      [ea / tot, eb / tot, jnp.zeros((y.shape[0], 6), _f32)], axis=1)


def _row_spec(width):
  return pl.BlockSpec((BLK, width), lambda i: (i, 0))


_deg_spec = pl.BlockSpec((NC, BLK, 8), lambda i: (0, i, 0))
_full = lambda shape: pl.BlockSpec(shape, lambda i: tuple(0 for _ in shape))

_prep1 = pl.pallas_call(
    _prep1_body,
    grid=(GRID,),
    in_specs=[_deg_spec, _row_spec(D_IN), _full((D_IN, D_HID))],
    out_specs=_row_spec(D_HID),
    out_shape=jax.ShapeDtypeStruct((N, D_HID), _f32),
)

_stats1 = pl.pallas_call(
    _stats1_body,
    grid=(GRID,),
    in_specs=[_deg_spec, pl.BlockSpec((NC, BLK, D_HID), lambda i: (0, i, 0)),
              _row_spec(D_HID), _full((1, D_HID))],
    out_specs=_full((8, D_HID)),
    out_shape=jax.ShapeDtypeStruct((8, D_HID), _f32),
)

_apply1 = pl.pallas_call(
    _apply1_body,
    grid=(GRID,),
    in_specs=[_deg_spec, pl.BlockSpec((NC, BLK, D_HID), lambda i: (0, i, 0)),
              _row_spec(D_HID), _full((1, D_HID)), _full((8, D_HID)),
              _full((1, D_HID)), _full((1, D_HID)), _full((D_HID, 8))],
    out_specs=_row_spec(8),
    out_shape=jax.ShapeDtypeStruct((N, 8), _f32),
)

_stats2 = pl.pallas_call(
    _stats2_body,
    grid=(GRID,),
    in_specs=[_deg_spec, pl.BlockSpec((NC, BLK, 8), lambda i: (0, i, 0)),
              _row_spec(8), _full((1, 8))],
    out_specs=_full((8, 8)),
    out_shape=jax.ShapeDtypeStruct((8, 8), _f32),
)

_apply2 = pl.pallas_call(
    _apply2_body,
    grid=(GRID,),
    in_specs=[_deg_spec, pl.BlockSpec((NC, BLK, 8), lambda i: (0, i, 0)),
              _row_spec(8), _full((1, 8)), _full((8, 8)),
              _full((1, 8)), _full((1, 8))],
    out_specs=[_row_spec(8), _row_spec(8)],
    out_shape=[jax.ShapeDtypeStruct((N, 8), _f32),
               jax.ShapeDtypeStruct((N, 8), _f32)],
)


@jax.jit
def kernel(edge_index, u_S, W1, b1, gamma1, beta1, W2, b2, gamma2, beta2):
  pad = EP - E
  src = jnp.concatenate(
      [edge_index[0], jnp.zeros((pad,), jnp.int32)]).reshape(EP // CH, CH)
  dst = jnp.concatenate(
      [edge_index[1], jnp.zeros((pad,), jnp.int32)]).reshape(EP // CH, CH)

  ones8 = jnp.ones((CH, 8), _f32)
  zeros8 = jnp.zeros((ZROWS, 8), _f32)
  zeros128 = jnp.zeros((ZROWS, D_HID), _f32)

  deg2 = _deg_kernel(dst, ones8, zeros8)

  w1 = W1.astype(_f32)
  g1 = _prep1(deg2, u_S, w1)
  acc1 = _prop128(src, dst, g1, zeros128)

  b1r = b1.reshape(1, D_HID)
  st1 = _stats1(deg2, acc1, g1, b1r)
  w2p = jnp.concatenate([W2, jnp.zeros((D_HID, 8 - D_OUT), _f32)], axis=1)
  g2 = _apply1(deg2, acc1, g1, b1r, st1, gamma1.reshape(1, -1),
               beta1.reshape(1, -1), w2p)

  acc2 = _prop8(src, dst, g2, zeros8)

  pad2 = lambda v: jnp.concatenate([v, jnp.zeros((8 - D_OUT,), _f32)]).reshape(1, 8)
  b2r = pad2(b2)
  st2 = _stats2(deg2, acc2, g2, b2r)
  sig, sm = _apply2(deg2, acc2, g2, b2r, st2, pad2(gamma2), pad2(beta2))
  return sig[:, :D_OUT], sm[:, :D_OUT]
